# baseline (speedup 1.0000x reference)
import functools

import jax
import jax.numpy as jnp
from jax.experimental import pallas as pl
from jax.experimental.pallas import tpu as pltpu


def _prep_kernel(wst_ref, bst_ref, wu_ref, bu_ref, wv_ref, bv_ref, m_ref,
                 wbig_ref, bias_ref, *, R, C):
    wv = wv_ref[...]
    rows = []
    brows = []
    for a in range(R):
        wu_a = wu_ref[a * C:(a + 1) * C, :]
        rows.append(jnp.dot(wst_ref[a], wu_a,
                            preferred_element_type=jnp.float32))
        brows.append(jnp.dot(bst_ref[a:a + 1, :], wu_a,
                             preferred_element_type=jnp.float32))
    weff = jnp.concatenate(rows, axis=0)
    wbig_ref[...] = jnp.dot(weff, wv, preferred_element_type=jnp.float32)
    bu_rows = jnp.concatenate(brows, axis=0)
    h = jnp.dot(m_ref[...], bu_rows,
                preferred_element_type=jnp.float32) + bu_ref[...]
    bias_ref[...] = jnp.dot(h, wv,
                            preferred_element_type=jnp.float32) + bv_ref[...]


def _mk(x_hbm, cm_ref, wst_ref, bst_ref, wu_ref, bu_ref, wv_ref, bv_ref,
        m_ref, o_hbm, xbuf, obuf, wbig_s, bias_s, insem, outsem,
        *, G, DEPTH, ODEPTH, NSTEPS, KR, KW, T, R, C):
    i = pl.program_id(0)
    TR = T // KR
    TW = T // KW

    def in_copies(step, slot):
        return [pltpu.make_async_copy(
                    x_hbm.at[pl.ds(step * G, G), pl.ds(k * TR, TR)],
                    xbuf.at[slot, slice(None), pl.ds(k * TR, TR)],
                    insem.at[slot, k])
                for k in range(KR)]

    def out_copies(step, slot):
        return [pltpu.make_async_copy(
                    obuf.at[slot, slice(None), pl.ds(k * TW, TW)],
                    o_hbm.at[pl.ds(step * G, G), pl.ds(k * TW, TW)],
                    outsem.at[slot, k])
                for k in range(KW)]

    @pl.when(i == 0)
    def _():
        for d in range(DEPTH):
            for c in in_copies(d, d):
                c.start()

    @pl.when(i == 0)
    def _():
        wv = wv_ref[...]
        rows = []
        brows = []
        for a in range(R):
            wu_a = wu_ref[a * C:(a + 1) * C, :]
            rows.append(jnp.dot(wst_ref[a], wu_a,
                                preferred_element_type=jnp.float32))
            brows.append(jnp.dot(bst_ref[a:a + 1, :], wu_a,
                                 preferred_element_type=jnp.float32))
        weff = jnp.concatenate(rows, axis=0)
        wbig_s[...] = jnp.dot(weff, wv, preferred_element_type=jnp.float32)
        bu_rows = jnp.concatenate(brows, axis=0)
        h = jnp.dot(m_ref[...], bu_rows,
                    preferred_element_type=jnp.float32) + bu_ref[...]
        bias_s[...] = jnp.dot(h, wv,
                              preferred_element_type=jnp.float32) + bv_ref[...]

    wbig = wbig_s[...]
    islot = jax.lax.rem(i, DEPTH)
    oslot = jax.lax.rem(i, ODEPTH)
    for c in in_copies(i, islot):
        c.wait()

    @pl.when(i >= ODEPTH)
    def _():
        for c in out_copies(i - ODEPTH, oslot):
            c.wait()

    for g in range(G):
        cm = cm_ref[pl.ds(i * G + g, 1), 0, :]
        bias = bias_s[pl.ds(i * G + g, 1), :]
        xz = xbuf[islot, g] * cm
        acc = jnp.dot(xz, wbig, preferred_element_type=jnp.float32)
        obuf[oslot, g] = acc + bias

    for c in out_copies(i, oslot):
        c.start()

    @pl.when(i + DEPTH < NSTEPS)
    def _():
        for c in in_copies(i + DEPTH, islot):
            c.start()

    @pl.when(i == NSTEPS - 1)
    def _():
        for d in range(ODEPTH):
            step = NSTEPS - ODEPTH + d
            for c in out_copies(step, jax.lax.rem(jnp.int32(step), ODEPTH)):
                c.wait()


@jax.jit
def kernel(spikes, neuron_regions, is_left, W_stitch, b_stitch, W_U, b_U,
           W_V, b_V):
    B, T, N = spikes.shape
    R, NPR, C = W_stitch.shape
    HIDDEN = W_U.shape[1]
    N_LAT = W_V.shape[1]
    R_kept = int(R * (1.0 - 0.25))

    noise = jax.random.uniform(jax.random.key(12345), (B, R))
    ids_restore = jnp.argsort(jnp.argsort(noise, axis=1), axis=1)
    m = (ids_restore < R_kept).astype(jnp.float32)
    colmask = jnp.repeat(m, NPR, axis=1)

    G = 4
    DEPTH = 3
    ODEPTH = 3
    KR = 2
    KW = 2
    NSTEPS = B // G
    out = pl.pallas_call(
        functools.partial(_mk, G=G, DEPTH=DEPTH, ODEPTH=ODEPTH,
                          NSTEPS=NSTEPS, KR=KR, KW=KW, T=T, R=R, C=C),
        grid=(NSTEPS,),
        in_specs=[
            pl.BlockSpec(memory_space=pltpu.MemorySpace.HBM),
            pl.BlockSpec((B, 1, N), lambda i: (0, 0, 0)),
            pl.BlockSpec((R, NPR, C), lambda i: (0, 0, 0)),
            pl.BlockSpec((R, C), lambda i: (0, 0)),
            pl.BlockSpec((R * C, HIDDEN), lambda i: (0, 0)),
            pl.BlockSpec((1, HIDDEN), lambda i: (0, 0)),
            pl.BlockSpec((HIDDEN, N_LAT), lambda i: (0, 0)),
            pl.BlockSpec((1, N_LAT), lambda i: (0, 0)),
            pl.BlockSpec((B, R), lambda i: (0, 0)),
        ],
        out_specs=pl.BlockSpec(memory_space=pltpu.MemorySpace.HBM),
        out_shape=jax.ShapeDtypeStruct((B, T, N_LAT), jnp.float32),
        scratch_shapes=[
            pltpu.VMEM((DEPTH, G, T, N), jnp.float32),
            pltpu.VMEM((ODEPTH, G, T, N_LAT), jnp.float32),
            pltpu.VMEM((N, N_LAT), jnp.float32),
            pltpu.VMEM((B, N_LAT), jnp.float32),
            pltpu.SemaphoreType.DMA((DEPTH, KR)),
            pltpu.SemaphoreType.DMA((ODEPTH, KW)),
        ],
        compiler_params=pltpu.CompilerParams(
            dimension_semantics=("arbitrary",)),
    )(spikes, colmask.reshape(B, 1, N), W_stitch, b_stitch, W_U,
      b_U.reshape(1, HIDDEN), W_V, b_V.reshape(1, N_LAT), m)
    return out


# speedup vs baseline: 1.0032x; 1.0032x over previous
import functools

import jax
import jax.numpy as jnp
from jax.experimental import pallas as pl
from jax.experimental.pallas import tpu as pltpu


def _prep_kernel(wst_ref, bst_ref, wu_ref, bu_ref, wv_ref, bv_ref, m_ref,
                 wbig_ref, bias_ref, *, R, C):
    wv = wv_ref[...]
    rows = []
    brows = []
    for a in range(R):
        wu_a = wu_ref[a * C:(a + 1) * C, :]
        rows.append(jnp.dot(wst_ref[a], wu_a,
                            preferred_element_type=jnp.float32))
        brows.append(jnp.dot(bst_ref[a:a + 1, :], wu_a,
                             preferred_element_type=jnp.float32))
    weff = jnp.concatenate(rows, axis=0)
    wbig_ref[...] = jnp.dot(weff, wv, preferred_element_type=jnp.float32)
    bu_rows = jnp.concatenate(brows, axis=0)
    h = jnp.dot(m_ref[...], bu_rows,
                preferred_element_type=jnp.float32) + bu_ref[...]
    bias_ref[...] = jnp.dot(h, wv,
                            preferred_element_type=jnp.float32) + bv_ref[...]


def _mk(x_hbm, cm_ref, wst_ref, bst_ref, wu_ref, bu_ref, wv_ref, bv_ref,
        m_ref, o_hbm, xbuf, obuf, wbig_s, bias_s, insem, outsem,
        *, G, DEPTH, ODEPTH, NSTEPS, KR, KW, T, R, C):
    i = pl.program_id(0)
    TR = T // KR
    TW = T // KW

    def in_copies(step, slot):
        return [pltpu.make_async_copy(
                    x_hbm.at[pl.ds(step * G, G), pl.ds(k * TR, TR)],
                    xbuf.at[slot, slice(None), pl.ds(k * TR, TR)],
                    insem.at[slot, k])
                for k in range(KR)]

    def out_copies(step, slot):
        return [pltpu.make_async_copy(
                    obuf.at[slot, slice(None), pl.ds(k * TW, TW)],
                    o_hbm.at[pl.ds(step * G, G), pl.ds(k * TW, TW)],
                    outsem.at[slot, k])
                for k in range(KW)]

    @pl.when(i == 0)
    def _():
        for d in range(DEPTH):
            for c in in_copies(d, d):
                c.start()

    @pl.when(i == 0)
    def _():
        wv = wv_ref[...]
        rows = []
        brows = []
        for a in range(R):
            wu_a = wu_ref[a * C:(a + 1) * C, :]
            rows.append(jnp.dot(wst_ref[a], wu_a,
                                preferred_element_type=jnp.float32))
            brows.append(jnp.dot(bst_ref[a:a + 1, :], wu_a,
                                 preferred_element_type=jnp.float32))
        weff = jnp.concatenate(rows, axis=0)
        wbig_s[...] = jnp.dot(weff, wv, preferred_element_type=jnp.float32)
        bu_rows = jnp.concatenate(brows, axis=0)
        h = jnp.dot(m_ref[...], bu_rows,
                    preferred_element_type=jnp.float32) + bu_ref[...]
        bias_s[...] = jnp.dot(h, wv,
                              preferred_element_type=jnp.float32) + bv_ref[...]

    wbig = wbig_s[...]
    islot = jax.lax.rem(i, DEPTH)
    oslot = jax.lax.rem(i, ODEPTH)
    for c in in_copies(i, islot):
        c.wait()

    @pl.when(i >= ODEPTH)
    def _():
        for c in out_copies(i - ODEPTH, oslot):
            c.wait()

    for g in range(G):
        cm = cm_ref[pl.ds(i * G + g, 1), 0, :]
        bias = bias_s[pl.ds(i * G + g, 1), :]
        xz = xbuf[islot, g] * cm
        acc = jnp.dot(xz, wbig, preferred_element_type=jnp.float32)
        obuf[oslot, g] = acc + bias

    for c in out_copies(i, oslot):
        c.start()

    @pl.when(i + DEPTH < NSTEPS)
    def _():
        for c in in_copies(i + DEPTH, islot):
            c.start()

    @pl.when(i == NSTEPS - 1)
    def _():
        for d in range(ODEPTH):
            step = NSTEPS - ODEPTH + d
            for c in out_copies(step, jax.lax.rem(jnp.int32(step), ODEPTH)):
                c.wait()


@jax.jit
def kernel(spikes, neuron_regions, is_left, W_stitch, b_stitch, W_U, b_U,
           W_V, b_V):
    B, T, N = spikes.shape
    R, NPR, C = W_stitch.shape
    HIDDEN = W_U.shape[1]
    N_LAT = W_V.shape[1]
    R_kept = int(R * (1.0 - 0.25))

    noise = jax.random.uniform(jax.random.key(12345), (B, R))
    ids_restore = jnp.argsort(jnp.argsort(noise, axis=1), axis=1)
    m = (ids_restore < R_kept).astype(jnp.float32)
    colmask = jnp.repeat(m, NPR, axis=1)

    G = 2
    DEPTH = 3
    ODEPTH = 3
    KR = 2
    KW = 2
    NSTEPS = B // G
    out = pl.pallas_call(
        functools.partial(_mk, G=G, DEPTH=DEPTH, ODEPTH=ODEPTH,
                          NSTEPS=NSTEPS, KR=KR, KW=KW, T=T, R=R, C=C),
        grid=(NSTEPS,),
        in_specs=[
            pl.BlockSpec(memory_space=pltpu.MemorySpace.HBM),
            pl.BlockSpec((B, 1, N), lambda i: (0, 0, 0)),
            pl.BlockSpec((R, NPR, C), lambda i: (0, 0, 0)),
            pl.BlockSpec((R, C), lambda i: (0, 0)),
            pl.BlockSpec((R * C, HIDDEN), lambda i: (0, 0)),
            pl.BlockSpec((1, HIDDEN), lambda i: (0, 0)),
            pl.BlockSpec((HIDDEN, N_LAT), lambda i: (0, 0)),
            pl.BlockSpec((1, N_LAT), lambda i: (0, 0)),
            pl.BlockSpec((B, R), lambda i: (0, 0)),
        ],
        out_specs=pl.BlockSpec(memory_space=pltpu.MemorySpace.HBM),
        out_shape=jax.ShapeDtypeStruct((B, T, N_LAT), jnp.float32),
        scratch_shapes=[
            pltpu.VMEM((DEPTH, G, T, N), jnp.float32),
            pltpu.VMEM((ODEPTH, G, T, N_LAT), jnp.float32),
            pltpu.VMEM((N, N_LAT), jnp.float32),
            pltpu.VMEM((B, N_LAT), jnp.float32),
            pltpu.SemaphoreType.DMA((DEPTH, KR)),
            pltpu.SemaphoreType.DMA((ODEPTH, KW)),
        ],
        compiler_params=pltpu.CompilerParams(
            dimension_semantics=("arbitrary",)),
    )(spikes, colmask.reshape(B, 1, N), W_stitch, b_stitch, W_U,
      b_U.reshape(1, HIDDEN), W_V, b_V.reshape(1, N_LAT), m)
    return out
